# baseline (speedup 1.0000x reference)
import functools

import jax
import jax.numpy as jnp
from jax import lax
from jax.experimental import pallas as pl
from jax.experimental.pallas import tpu as pltpu
from jax.experimental.pallas import tpu_sc as plsc

NC, NS = 2, 16
NW = NC * NS
SEQ = 200
EMB = 16
VOCAB = 1000003
VC = 1024
NVCHUNK = VOCAB // VC
VMAIN = NVCHUNK * VC
VTAIL = VOCAB - VMAIN


def _transpose_body(embT_hbm, tail_hbm, lin_hbm, in_v, out_v, tail_v):
    wid = lax.axis_index("s") * NC + lax.axis_index("c")
    nk = jnp.where(wid < NVCHUNK % NW, NVCHUNK // NW + 1, NVCHUNK // NW)
    iot = lax.iota(jnp.int32, 16)

    def chunk_body(k, c):
        v0 = (wid + k * NW) * VC
        pltpu.sync_copy(embT_hbm.at[:, pl.ds(v0, VC)], in_v)

        def vbody(v, c2):
            out_v[pl.ds(v * EMB, EMB)] = plsc.load_gather(
                in_v, [iot, jnp.full((16,), v, jnp.int32)]
            )
            return c2

        lax.fori_loop(0, VC, vbody, 0)
        pltpu.sync_copy(out_v, lin_hbm.at[pl.ds(v0 * EMB, VC * EMB)])
        return c

    lax.fori_loop(0, nk, chunk_body, 0)

    @pl.when(wid == 0)
    def _():
        pltpu.sync_copy(tail_hbm, tail_v)

        def vbody(v, c):
            out_v[pl.ds(v * EMB, EMB)] = tail_v[v, :]
            return c

        lax.fori_loop(0, VTAIL, vbody, 0)
        pltpu.sync_copy(out_v.at[pl.ds(0, VTAIL * EMB)],
                        lin_hbm.at[pl.ds(VMAIN * EMB, VTAIL * EMB)])


def _lookup_body(seq, chunk, nchunk, per_w,
                 tok_hbm, map1_hbm, map2_hbm, emb_hbm, pe_hbm, out_hbm,
                 tok_v, t1_v, t2_v, rows1_v, rows2_v, out_v, pe_v,
                 sem_m, sem_e):
    wid = lax.axis_index("s") * NC + lax.axis_index("c")
    pltpu.sync_copy(pe_hbm.at[pl.ds(0, seq)], pe_v)

    def chunk_body(g, carry):
        base = wid * per_w + g * chunk
        pltpu.sync_copy(tok_hbm.at[pl.ds(base, chunk)], tok_v)
        c1 = pltpu.async_copy(map1_hbm.at[tok_v], t1_v, sem_m)
        c2 = pltpu.async_copy(map2_hbm.at[tok_v], t2_v, sem_m)
        c1.wait()
        c2.wait()
        d1 = pltpu.async_copy(emb_hbm.at[t1_v], rows1_v, sem_e)
        d2 = pltpu.async_copy(emb_hbm.at[t2_v], rows2_v, sem_e)
        d1.wait()
        d2.wait()

        def seq_body(s, c2_):
            def pos_body(p, c3_):
                i = s * seq + p
                out_v[i, :] = (rows1_v[i, :] + rows2_v[i, :]) * 2.0 + pe_v[p, :]
                return c3_

            return lax.fori_loop(0, seq, pos_body, c2_)

        lax.fori_loop(0, chunk // seq, seq_body, 0)
        pltpu.sync_copy(out_v, out_hbm.at[pl.ds(base, chunk)])
        return carry

    lax.fori_loop(0, nchunk, chunk_body, 0)


def kernel(tokens, embedding, positional_embedding, mapping1, mapping2):
    bsz, seqlen = tokens.shape
    n = bsz * seqlen
    assert seqlen == SEQ and n % NW == 0 and embedding.shape == (VOCAB, EMB)
    per_w = n // NW
    chunk = 1600
    assert per_w % chunk == 0
    nchunk = per_w // chunk

    mesh = plsc.VectorSubcoreMesh(core_axis_name="c", subcore_axis_name="s")

    lin_flat = pl.kernel(
        _transpose_body,
        out_type=jax.ShapeDtypeStruct((VOCAB * EMB,), jnp.float32),
        mesh=mesh,
        compiler_params=pltpu.CompilerParams(
            use_tc_tiling_on_sc=True, needs_layout_passes=False),
        scratch_types=[
            pltpu.VMEM((EMB, VC), jnp.float32),
            pltpu.VMEM((VC * EMB,), jnp.float32),
            pltpu.VMEM((VTAIL, EMB), jnp.float32),
        ],
    )(embedding.T, embedding[VMAIN:])
    emb_lin = lin_flat.reshape(VOCAB, EMB)

    tok = tokens.reshape(n).astype(jnp.int32)
    pe = positional_embedding.reshape(-1, EMB)

    body = functools.partial(_lookup_body, seqlen, chunk, nchunk, per_w)
    out = pl.kernel(
        body,
        out_type=jax.ShapeDtypeStruct((n, EMB), jnp.float32),
        mesh=mesh,
        compiler_params=pltpu.CompilerParams(use_tc_tiling_on_sc=False),
        scratch_types=[
            pltpu.VMEM((chunk,), jnp.int32),
            pltpu.VMEM((chunk,), jnp.int32),
            pltpu.VMEM((chunk,), jnp.int32),
            pltpu.VMEM((chunk, EMB), jnp.float32),
            pltpu.VMEM((chunk, EMB), jnp.float32),
            pltpu.VMEM((chunk, EMB), jnp.float32),
            pltpu.VMEM((SEQ, EMB), jnp.float32),
            pltpu.SemaphoreType.DMA,
            pltpu.SemaphoreType.DMA,
        ],
    )(tok, mapping1, mapping2, emb_lin, pe)
    return out.reshape(bsz, seqlen, EMB)


# speedup vs baseline: 1.3240x; 1.3240x over previous
import functools

import jax
import jax.numpy as jnp
from jax import lax
from jax.experimental import pallas as pl
from jax.experimental.pallas import tpu as pltpu
from jax.experimental.pallas import tpu_sc as plsc

NC, NS = 2, 16
NW = NC * NS
SEQ = 200
EMB = 16
VOCAB = 1000003
VC = 1024
NVCHUNK = VOCAB // VC
VMAIN = NVCHUNK * VC
VTAIL = VOCAB - VMAIN


def _transpose_body(embT_hbm, tail_hbm, lin_hbm, in_v, out_v, tail_v):
    wid = lax.axis_index("s") * NC + lax.axis_index("c")
    nk = jnp.where(wid < NVCHUNK % NW, NVCHUNK // NW + 1, NVCHUNK // NW)
    iot = lax.iota(jnp.int32, 16)

    def chunk_body(k, c):
        v0 = (wid + k * NW) * VC
        pltpu.sync_copy(embT_hbm.at[:, pl.ds(v0, VC)], in_v)

        @plsc.parallel_loop(0, VC, 1, unroll=8)
        def _(v):
            out_v[pl.ds(v * EMB, EMB)] = plsc.load_gather(
                in_v, [iot, jnp.full((16,), v, jnp.int32)]
            )
        pltpu.sync_copy(out_v, lin_hbm.at[pl.ds(v0 * EMB, VC * EMB)])
        return c

    lax.fori_loop(0, nk, chunk_body, 0)

    @pl.when(wid == 0)
    def _():
        pltpu.sync_copy(tail_hbm, tail_v)

        def vbody(v, c):
            out_v[pl.ds(v * EMB, EMB)] = tail_v[v, :]
            return c

        lax.fori_loop(0, VTAIL, vbody, 0)
        pltpu.sync_copy(out_v.at[pl.ds(0, VTAIL * EMB)],
                        lin_hbm.at[pl.ds(VMAIN * EMB, VTAIL * EMB)])


def _lookup_body(seq, chunk, nchunk, per_w,
                 tok_hbm, map1_hbm, map2_hbm, emb_hbm, pe_hbm, out_hbm,
                 tok_v, t1_v, t2_v, rows1_v, rows2_v, out_v, pe_v,
                 sem_m, sem_e):
    wid = lax.axis_index("s") * NC + lax.axis_index("c")
    for r in range(chunk // seq):
        pltpu.sync_copy(pe_hbm.at[pl.ds(0, seq)], pe_v.at[pl.ds(r * seq, seq)])

    def chunk_body(g, carry):
        base = wid * per_w + g * chunk
        pltpu.sync_copy(tok_hbm.at[pl.ds(base, chunk)], tok_v)
        c1 = pltpu.async_copy(map1_hbm.at[tok_v], t1_v, sem_m)
        c2 = pltpu.async_copy(map2_hbm.at[tok_v], t2_v, sem_m)
        c1.wait()
        c2.wait()
        d1 = pltpu.async_copy(emb_hbm.at[t1_v], rows1_v, sem_e)
        d2 = pltpu.async_copy(emb_hbm.at[t2_v], rows2_v, sem_e)
        d1.wait()
        d2.wait()

        @plsc.parallel_loop(0, chunk, 1, unroll=8)
        def _(i):
            out_v[i, :] = (rows1_v[i, :] + rows2_v[i, :]) * 2.0 + pe_v[i, :]

        pltpu.sync_copy(out_v, out_hbm.at[pl.ds(base, chunk)])
        return carry

    lax.fori_loop(0, nchunk, chunk_body, 0)


def kernel(tokens, embedding, positional_embedding, mapping1, mapping2):
    bsz, seqlen = tokens.shape
    n = bsz * seqlen
    assert seqlen == SEQ and n % NW == 0 and embedding.shape == (VOCAB, EMB)
    per_w = n // NW
    chunk = 1600
    assert per_w % chunk == 0
    nchunk = per_w // chunk

    mesh = plsc.VectorSubcoreMesh(core_axis_name="c", subcore_axis_name="s")

    lin_flat = pl.kernel(
        _transpose_body,
        out_type=jax.ShapeDtypeStruct((VOCAB * EMB,), jnp.float32),
        mesh=mesh,
        compiler_params=pltpu.CompilerParams(
            use_tc_tiling_on_sc=True, needs_layout_passes=False),
        scratch_types=[
            pltpu.VMEM((EMB, VC), jnp.float32),
            pltpu.VMEM((VC * EMB,), jnp.float32),
            pltpu.VMEM((VTAIL, EMB), jnp.float32),
        ],
    )(embedding.T, embedding[VMAIN:])
    emb_lin = lin_flat.reshape(VOCAB, EMB)

    tok = tokens.reshape(n).astype(jnp.int32)
    pe = positional_embedding.reshape(-1, EMB)

    body = functools.partial(_lookup_body, seqlen, chunk, nchunk, per_w)
    out = pl.kernel(
        body,
        out_type=jax.ShapeDtypeStruct((n, EMB), jnp.float32),
        mesh=mesh,
        compiler_params=pltpu.CompilerParams(use_tc_tiling_on_sc=False),
        scratch_types=[
            pltpu.VMEM((chunk,), jnp.int32),
            pltpu.VMEM((chunk,), jnp.int32),
            pltpu.VMEM((chunk,), jnp.int32),
            pltpu.VMEM((chunk, EMB), jnp.float32),
            pltpu.VMEM((chunk, EMB), jnp.float32),
            pltpu.VMEM((chunk, EMB), jnp.float32),
            pltpu.VMEM((chunk, EMB), jnp.float32),
            pltpu.SemaphoreType.DMA,
            pltpu.SemaphoreType.DMA,
        ],
    )(tok, mapping1, mapping2, emb_lin, pe)
    return out.reshape(bsz, seqlen, EMB)


# speedup vs baseline: 1.6648x; 1.2575x over previous
import functools

import jax
import jax.numpy as jnp
from jax import lax
from jax.experimental import pallas as pl
from jax.experimental.pallas import tpu as pltpu
from jax.experimental.pallas import tpu_sc as plsc

NC, NS = 2, 16
NW = NC * NS
SEQ = 200
EMB = 16
VOCAB = 1000003
VC = 1024
NVCHUNK = VOCAB // VC
VMAIN = NVCHUNK * VC
VTAIL = VOCAB - VMAIN


def _transpose_body(embT_hbm, tail_hbm, lin_hbm, in_v, out_v, tail_v):
    wid = lax.axis_index("s") * NC + lax.axis_index("c")
    nk = jnp.where(wid < NVCHUNK % NW, NVCHUNK // NW + 1, NVCHUNK // NW)
    iot = lax.iota(jnp.int32, 16)

    def chunk_body(k, c):
        v0 = (wid + k * NW) * VC
        pltpu.sync_copy(embT_hbm.at[:, pl.ds(v0, VC)], in_v)

        @plsc.parallel_loop(0, VC, 1, unroll=8)
        def _(v):
            out_v[pl.ds(v * EMB, EMB)] = plsc.load_gather(
                in_v, [iot, jnp.full((16,), v, jnp.int32)]
            )
        pltpu.sync_copy(out_v, lin_hbm.at[pl.ds(v0 * EMB, VC * EMB)])
        return c

    lax.fori_loop(0, nk, chunk_body, 0)

    @pl.when(wid == 0)
    def _():
        pltpu.sync_copy(tail_hbm, tail_v)

        def vbody(v, c):
            out_v[pl.ds(v * EMB, EMB)] = tail_v[v, :]
            return c

        lax.fori_loop(0, VTAIL, vbody, 0)
        pltpu.sync_copy(out_v.at[pl.ds(0, VTAIL * EMB)],
                        lin_hbm.at[pl.ds(VMAIN * EMB, VTAIL * EMB)])


SC_S = 10
BW = 128


def _lookup_body(tokT_hbm, map1_hbm, map2_hbm, emb_hbm, pe_hbm, out_hbm,
                 tok_v, t1_v, t2_v, rows1_v, rows2_v, out_v, pe_v,
                 sem_m, sem_e):
    wid = lax.axis_index("s") * NC + lax.axis_index("c")
    b0 = wid * BW
    pltpu.sync_copy(pe_hbm.at[pl.ds(0, SEQ)], pe_v)
    iot = lax.iota(jnp.int32, 16)

    def chunk_body(g, carry):
        s0 = g * SC_S
        pltpu.sync_copy(tokT_hbm.at[pl.ds(s0, SC_S), pl.ds(b0, BW)], tok_v)
        cs = []
        for sl in range(SC_S):
            cs.append(pltpu.async_copy(
                map1_hbm.at[tok_v.at[sl]], t1_v.at[sl], sem_m))
            cs.append(pltpu.async_copy(
                map2_hbm.at[tok_v.at[sl]], t2_v.at[sl], sem_m))
        for c in cs:
            c.wait()
        ds_ = []
        for sl in range(SC_S):
            ds_.append(pltpu.async_copy(
                emb_hbm.at[t1_v.at[sl]], rows1_v.at[sl], sem_e))
            ds_.append(pltpu.async_copy(
                emb_hbm.at[t2_v.at[sl]], rows2_v.at[sl], sem_e))
        for d in ds_:
            d.wait()

        @plsc.parallel_loop(0, SC_S * EMB * (BW // 16), 1, unroll=4)
        def _(i):
            sl = i // (EMB * (BW // 16))
            r = i % (EMB * (BW // 16))
            d = r // (BW // 16)
            bv = r % (BW // 16)
            bidx = iot + bv * 16
            g1 = plsc.load_gather(
                rows1_v, [jnp.full((16,), sl, jnp.int32), bidx,
                          jnp.full((16,), d, jnp.int32)])
            g2 = plsc.load_gather(
                rows2_v, [jnp.full((16,), sl, jnp.int32), bidx,
                          jnp.full((16,), d, jnp.int32)])
            pes = plsc.load_gather(
                pe_v, [jnp.full((16,), s0 + sl, jnp.int32),
                       jnp.full((16,), d, jnp.int32)])
            out_v[sl, d, pl.ds(bv * 16, 16)] = (g1 + g2) * 2.0 + pes

        pltpu.sync_copy(out_v,
                        out_hbm.at[pl.ds(s0, SC_S), :, pl.ds(b0, BW)])
        return carry

    lax.fori_loop(0, SEQ // SC_S, chunk_body, 0)


def kernel(tokens, embedding, positional_embedding, mapping1, mapping2):
    bsz, seqlen = tokens.shape
    assert seqlen == SEQ and bsz == NW * BW
    assert embedding.shape == (VOCAB, EMB)

    mesh = plsc.VectorSubcoreMesh(core_axis_name="c", subcore_axis_name="s")

    lin_flat = pl.kernel(
        _transpose_body,
        out_type=jax.ShapeDtypeStruct((VOCAB * EMB,), jnp.float32),
        mesh=mesh,
        compiler_params=pltpu.CompilerParams(
            use_tc_tiling_on_sc=True, needs_layout_passes=False),
        scratch_types=[
            pltpu.VMEM((EMB, VC), jnp.float32),
            pltpu.VMEM((VC * EMB,), jnp.float32),
            pltpu.VMEM((VTAIL, EMB), jnp.float32),
        ],
    )(embedding.T, embedding[VMAIN:])
    emb_lin = lin_flat.reshape(VOCAB, EMB)

    tokT = tokens.T.astype(jnp.int32)
    pe = positional_embedding.reshape(-1, EMB)

    out = pl.kernel(
        _lookup_body,
        out_type=jax.ShapeDtypeStruct((SEQ, EMB, bsz), jnp.float32),
        mesh=mesh,
        compiler_params=pltpu.CompilerParams(
            use_tc_tiling_on_sc=False, needs_layout_passes=False),
        scratch_types=[
            pltpu.VMEM((SC_S, BW), jnp.int32),
            pltpu.VMEM((SC_S, BW), jnp.int32),
            pltpu.VMEM((SC_S, BW), jnp.int32),
            pltpu.VMEM((SC_S, BW, EMB), jnp.float32),
            pltpu.VMEM((SC_S, BW, EMB), jnp.float32),
            pltpu.VMEM((SC_S, EMB, BW), jnp.float32),
            pltpu.VMEM((SEQ, EMB), jnp.float32),
            pltpu.SemaphoreType.DMA,
            pltpu.SemaphoreType.DMA,
        ],
    )(tokT, mapping1, mapping2, emb_lin, pe)
    return out.transpose(2, 0, 1)


# speedup vs baseline: 1.9289x; 1.1586x over previous
import functools

import jax
import jax.numpy as jnp
from jax import lax
from jax.experimental import pallas as pl
from jax.experimental.pallas import tpu as pltpu
from jax.experimental.pallas import tpu_sc as plsc

NC, NS = 2, 16
NW = NC * NS
SEQ = 200
EMB = 16
VOCAB = 1000003
VC = 1024
NVCHUNK = VOCAB // VC
VMAIN = NVCHUNK * VC
VTAIL = VOCAB - VMAIN


def _transpose_body(embT_hbm, tail_hbm, lin_hbm, in_v, out_v, tail_v):
    wid = lax.axis_index("s") * NC + lax.axis_index("c")
    nk = jnp.where(wid < NVCHUNK % NW, NVCHUNK // NW + 1, NVCHUNK // NW)
    iot = lax.iota(jnp.int32, 16)

    def chunk_body(k, c):
        v0 = (wid + k * NW) * VC
        pltpu.sync_copy(embT_hbm.at[:, pl.ds(v0, VC)], in_v)

        @plsc.parallel_loop(0, VC, 1, unroll=8)
        def _(v):
            out_v[pl.ds(v * EMB, EMB)] = plsc.load_gather(
                in_v, [iot, jnp.full((16,), v, jnp.int32)]
            )
        pltpu.sync_copy(out_v, lin_hbm.at[pl.ds(v0 * EMB, VC * EMB)])
        return c

    lax.fori_loop(0, nk, chunk_body, 0)

    @pl.when(wid == 0)
    def _():
        pltpu.sync_copy(tail_hbm, tail_v)

        def vbody(v, c):
            out_v[pl.ds(v * EMB, EMB)] = tail_v[v, :]
            return c

        lax.fori_loop(0, VTAIL, vbody, 0)
        pltpu.sync_copy(out_v.at[pl.ds(0, VTAIL * EMB)],
                        lin_hbm.at[pl.ds(VMAIN * EMB, VTAIL * EMB)])


NSG = 8
NBG = 4
S_PER = SEQ // NSG
BG = 4096 // NBG


def _lookup_body(tokT_hbm, map1_hbm, map2_hbm, emb_hbm, pe_hbm, out_hbm,
                 tok_v, t1_v, t2_v, rows1_v, rows2_v, out_v, pe_v,
                 sem_m, sem_e0, sem_e1):
    wid = lax.axis_index("s") * NC + lax.axis_index("c")
    sg = wid // NBG
    bg = wid % NBG
    b0 = bg * BG
    pltpu.sync_copy(pe_hbm.at[pl.ds(sg * S_PER, S_PER)], pe_v)
    iot = lax.iota(jnp.int32, 16)
    sems = (sem_e0, sem_e1)

    def issue_tok_maps(g):
        st = g % 2
        pltpu.sync_copy(tokT_hbm.at[sg * S_PER + g, pl.ds(b0, BG)],
                        tok_v.at[st])
        return (pltpu.async_copy(map1_hbm.at[tok_v.at[st]],
                                 t1_v.at[st], sem_m),
                pltpu.async_copy(map2_hbm.at[tok_v.at[st]],
                                 t2_v.at[st], sem_m))

    def issue_rows(g):
        st = g % 2
        return (pltpu.async_copy(emb_hbm.at[t1_v.at[st]],
                                 rows1_v.at[st], sems[st]),
                pltpu.async_copy(emb_hbm.at[t2_v.at[st]],
                                 rows2_v.at[st], sems[st]))

    def compute_store(g):
        st = g % 2
        st_c = jnp.int32(st)

        def dbody(d, c):
            pes = plsc.load_gather(
                pe_v, [jnp.full((16,), g, jnp.int32),
                       jnp.full((16,), d, jnp.int32)])

            @plsc.parallel_loop(0, BG // 16, 1, unroll=4)
            def _(bv):
                bidx = iot + bv * 16
                dd = jnp.full((16,), d, jnp.int32)
                ss = jnp.full((16,), st_c, jnp.int32)
                g1 = plsc.load_gather(rows1_v, [ss, bidx, dd])
                g2 = plsc.load_gather(rows2_v, [ss, bidx, dd])
                out_v[d, pl.ds(bv * 16, 16)] = (g1 + g2) * 2.0 + pes

            return c

        lax.fori_loop(0, EMB, dbody, 0)
        pltpu.sync_copy(out_v,
                        out_hbm.at[sg * S_PER + g, :, pl.ds(b0, BG)])

    maps = {0: issue_tok_maps(0)}
    for c in maps[0]:
        c.wait()
    rows = {0: issue_rows(0)}
    maps[1] = issue_tok_maps(1)
    for g in range(S_PER):
        if g + 1 < S_PER:
            for c in maps[g + 1]:
                c.wait()
            rows[g + 1] = issue_rows(g + 1)
        if g + 2 < S_PER:
            maps[g + 2] = issue_tok_maps(g + 2)
        for c in rows[g]:
            c.wait()
        compute_store(g)


def kernel(tokens, embedding, positional_embedding, mapping1, mapping2):
    bsz, seqlen = tokens.shape
    assert seqlen == SEQ and bsz == NBG * BG
    assert embedding.shape == (VOCAB, EMB)

    mesh = plsc.VectorSubcoreMesh(core_axis_name="c", subcore_axis_name="s")

    lin_flat = pl.kernel(
        _transpose_body,
        out_type=jax.ShapeDtypeStruct((VOCAB * EMB,), jnp.float32),
        mesh=mesh,
        compiler_params=pltpu.CompilerParams(
            use_tc_tiling_on_sc=True, needs_layout_passes=False),
        scratch_types=[
            pltpu.VMEM((EMB, VC), jnp.float32),
            pltpu.VMEM((VC * EMB,), jnp.float32),
            pltpu.VMEM((VTAIL, EMB), jnp.float32),
        ],
    )(embedding.T, embedding[VMAIN:])
    emb_lin = lin_flat.reshape(VOCAB, EMB)

    tokT = tokens.T.astype(jnp.int32)
    pe = positional_embedding.reshape(-1, EMB)

    out = pl.kernel(
        _lookup_body,
        out_type=jax.ShapeDtypeStruct((SEQ, EMB, bsz), jnp.float32),
        mesh=mesh,
        compiler_params=pltpu.CompilerParams(
            use_tc_tiling_on_sc=False, needs_layout_passes=False),
        scratch_types=[
            pltpu.VMEM((2, BG), jnp.int32),
            pltpu.VMEM((2, BG), jnp.int32),
            pltpu.VMEM((2, BG), jnp.int32),
            pltpu.VMEM((2, BG, EMB), jnp.float32),
            pltpu.VMEM((2, BG, EMB), jnp.float32),
            pltpu.VMEM((EMB, BG), jnp.float32),
            pltpu.VMEM((S_PER, EMB), jnp.float32),
            pltpu.SemaphoreType.DMA,
            pltpu.SemaphoreType.DMA,
            pltpu.SemaphoreType.DMA,
        ],
    )(tokT, mapping1, mapping2, emb_lin, pe)
    return out.transpose(2, 0, 1)


# speedup vs baseline: 2.0120x; 1.0431x over previous
import functools

import jax
import jax.numpy as jnp
from jax import lax
from jax.experimental import pallas as pl
from jax.experimental.pallas import tpu as pltpu
from jax.experimental.pallas import tpu_sc as plsc

NC, NS = 2, 16
NW = NC * NS
SEQ = 200
EMB = 16
VOCAB = 1000003
VC = 1024
NVCHUNK = VOCAB // VC
VMAIN = NVCHUNK * VC
VTAIL = VOCAB - VMAIN


NPAIR = NVCHUNK // 2


def _transpose_body(embT_hbm, tail_hbm, lin_hbm,
                    in0_v, in1_v, out0_v, out1_v, tail_v,
                    sem_r, sem_w0, sem_w1):
    wid = lax.axis_index("s") * NC + lax.axis_index("c")
    iot = lax.iota(jnp.int32, 16)

    def transpose(in_v, out_v):
        @plsc.parallel_loop(0, VC, 1, unroll=8)
        def _(v):
            out_v[pl.ds(v * EMB, EMB)] = plsc.load_gather(
                in_v, [iot, jnp.full((16,), v, jnp.int32)]
            )

    def pair_body(k, c):
        p = wid + k * NW

        @pl.when(p < NPAIR)
        def _():
            va = 2 * p * VC
            vb = va + VC

            @pl.when(k > 0)
            def _():
                pltpu.make_async_copy(
                    out0_v, lin_hbm.at[pl.ds(va * EMB, VC * EMB)],
                    sem_w0).wait()
                pltpu.make_async_copy(
                    out1_v, lin_hbm.at[pl.ds(va * EMB, VC * EMB)],
                    sem_w1).wait()

            pltpu.sync_copy(embT_hbm.at[:, pl.ds(va, VC)], in0_v)
            rb = pltpu.async_copy(embT_hbm.at[:, pl.ds(vb, VC)], in1_v,
                                  sem_r)
            transpose(in0_v, out0_v)
            pltpu.async_copy(out0_v, lin_hbm.at[pl.ds(va * EMB, VC * EMB)],
                             sem_w0)
            rb.wait()
            transpose(in1_v, out1_v)
            pltpu.async_copy(out1_v, lin_hbm.at[pl.ds(vb * EMB, VC * EMB)],
                             sem_w1)

        return c

    lax.fori_loop(0, (NPAIR + NW - 1) // NW, pair_body, 0)
    pltpu.make_async_copy(
        out0_v, lin_hbm.at[pl.ds(0, VC * EMB)], sem_w0).wait()
    pltpu.make_async_copy(
        out1_v, lin_hbm.at[pl.ds(0, VC * EMB)], sem_w1).wait()

    @pl.when(wid == 0)
    def _():
        pltpu.sync_copy(tail_hbm, tail_v)
        pltpu.sync_copy(tail_v,
                        lin_hbm.at[pl.ds(VMAIN * EMB, VTAIL * EMB)])


NSG = 8
NBG = 4
S_PER = SEQ // NSG
BG = 4096 // NBG


def _lookup_body(tokT_hbm, map1_hbm, map2_hbm, emb_hbm, pe_hbm, out_hbm,
                 tok_v, t1_v, t2_v, rows1_v, rows2_v, out_v, pe_v,
                 sem_m, sem_e0, sem_e1):
    wid = lax.axis_index("s") * NC + lax.axis_index("c")
    sg = wid // NBG
    bg = wid % NBG
    b0 = bg * BG
    pltpu.sync_copy(pe_hbm.at[pl.ds(sg * S_PER, S_PER)], pe_v)
    iot = lax.iota(jnp.int32, 16)
    sems = (sem_e0, sem_e1)

    def issue_tok_maps(g):
        st = g % 2
        pltpu.sync_copy(tokT_hbm.at[sg * S_PER + g, pl.ds(b0, BG)],
                        tok_v.at[st])
        return (pltpu.async_copy(map1_hbm.at[tok_v.at[st]],
                                 t1_v.at[st], sem_m),
                pltpu.async_copy(map2_hbm.at[tok_v.at[st]],
                                 t2_v.at[st], sem_m))

    def issue_rows(g):
        st = g % 2
        return (pltpu.async_copy(emb_hbm.at[t1_v.at[st]],
                                 rows1_v.at[st], sems[st]),
                pltpu.async_copy(emb_hbm.at[t2_v.at[st]],
                                 rows2_v.at[st], sems[st]))

    def compute_store(g):
        st = g % 2
        st_c = jnp.int32(st)

        def dbody(d, c):
            pes = plsc.load_gather(
                pe_v, [jnp.full((16,), g, jnp.int32),
                       jnp.full((16,), d, jnp.int32)])

            @plsc.parallel_loop(0, BG // 16, 1, unroll=4)
            def _(bv):
                bidx = iot + bv * 16
                dd = jnp.full((16,), d, jnp.int32)
                ss = jnp.full((16,), st_c, jnp.int32)
                g1 = plsc.load_gather(rows1_v, [ss, bidx, dd])
                g2 = plsc.load_gather(rows2_v, [ss, bidx, dd])
                out_v[d, pl.ds(bv * 16, 16)] = (g1 + g2) * 2.0 + pes

            return c

        lax.fori_loop(0, EMB, dbody, 0)
        pltpu.sync_copy(out_v,
                        out_hbm.at[sg * S_PER + g, :, pl.ds(b0, BG)])

    maps = {0: issue_tok_maps(0)}
    for c in maps[0]:
        c.wait()
    rows = {0: issue_rows(0)}
    maps[1] = issue_tok_maps(1)
    for g in range(S_PER):
        if g + 1 < S_PER:
            for c in maps[g + 1]:
                c.wait()
            rows[g + 1] = issue_rows(g + 1)
        if g + 2 < S_PER:
            maps[g + 2] = issue_tok_maps(g + 2)
        for c in rows[g]:
            c.wait()
        compute_store(g)


def kernel(tokens, embedding, positional_embedding, mapping1, mapping2):
    bsz, seqlen = tokens.shape
    assert seqlen == SEQ and bsz == NBG * BG
    assert embedding.shape == (VOCAB, EMB)

    mesh = plsc.VectorSubcoreMesh(core_axis_name="c", subcore_axis_name="s")

    lin_flat = pl.kernel(
        _transpose_body,
        out_type=jax.ShapeDtypeStruct((VOCAB * EMB,), jnp.float32),
        mesh=mesh,
        compiler_params=pltpu.CompilerParams(
            use_tc_tiling_on_sc=True, needs_layout_passes=False),
        scratch_types=[
            pltpu.VMEM((EMB, VC), jnp.float32),
            pltpu.VMEM((EMB, VC), jnp.float32),
            pltpu.VMEM((VC * EMB,), jnp.float32),
            pltpu.VMEM((VC * EMB,), jnp.float32),
            pltpu.VMEM((VTAIL * EMB,), jnp.float32),
            pltpu.SemaphoreType.DMA,
            pltpu.SemaphoreType.DMA,
            pltpu.SemaphoreType.DMA,
        ],
    )(embedding.T, embedding[VMAIN:].reshape(VTAIL * EMB))
    emb_lin = lin_flat.reshape(VOCAB, EMB)

    tokT = tokens.T.astype(jnp.int32)
    pe = positional_embedding.reshape(-1, EMB)

    out = pl.kernel(
        _lookup_body,
        out_type=jax.ShapeDtypeStruct((SEQ, EMB, bsz), jnp.float32),
        mesh=mesh,
        compiler_params=pltpu.CompilerParams(
            use_tc_tiling_on_sc=False, needs_layout_passes=False),
        scratch_types=[
            pltpu.VMEM((2, BG), jnp.int32),
            pltpu.VMEM((2, BG), jnp.int32),
            pltpu.VMEM((2, BG), jnp.int32),
            pltpu.VMEM((2, BG, EMB), jnp.float32),
            pltpu.VMEM((2, BG, EMB), jnp.float32),
            pltpu.VMEM((EMB, BG), jnp.float32),
            pltpu.VMEM((S_PER, EMB), jnp.float32),
            pltpu.SemaphoreType.DMA,
            pltpu.SemaphoreType.DMA,
            pltpu.SemaphoreType.DMA,
        ],
    )(tokT, mapping1, mapping2, emb_lin, pe)
    return out.transpose(2, 0, 1)


# speedup vs baseline: 2.0475x; 1.0177x over previous
import functools

import jax
import jax.numpy as jnp
from jax import lax
from jax.experimental import pallas as pl
from jax.experimental.pallas import tpu as pltpu
from jax.experimental.pallas import tpu_sc as plsc

NC, NS = 2, 16
NW = NC * NS
SEQ = 200
EMB = 16
VOCAB = 1000003
VC = 1024
NVCHUNK = VOCAB // VC
VMAIN = NVCHUNK * VC
VTAIL = VOCAB - VMAIN


NPAIR = NVCHUNK // 2


def _transpose_body(embT_hbm, tail_hbm, lin_hbm,
                    in0_v, in1_v, out0_v, out1_v, tail_v,
                    sem_r, sem_w0, sem_w1):
    wid = lax.axis_index("s") * NC + lax.axis_index("c")
    iot = lax.iota(jnp.int32, 16)

    def transpose(in_v, out_v):
        @plsc.parallel_loop(0, VC, 1, unroll=16)
        def _(v):
            out_v[pl.ds(v * EMB, EMB)] = plsc.load_gather(
                in_v, [iot, jnp.full((16,), v, jnp.int32)]
            )

    def pair_body(k, c):
        p = wid + k * NW

        @pl.when(p < NPAIR)
        def _():
            va = 2 * p * VC
            vb = va + VC

            @pl.when(k > 0)
            def _():
                pltpu.make_async_copy(
                    out0_v, lin_hbm.at[pl.ds(va * EMB, VC * EMB)],
                    sem_w0).wait()
                pltpu.make_async_copy(
                    out1_v, lin_hbm.at[pl.ds(va * EMB, VC * EMB)],
                    sem_w1).wait()

            pltpu.sync_copy(embT_hbm.at[:, pl.ds(va, VC)], in0_v)
            rb = pltpu.async_copy(embT_hbm.at[:, pl.ds(vb, VC)], in1_v,
                                  sem_r)
            transpose(in0_v, out0_v)
            pltpu.async_copy(out0_v, lin_hbm.at[pl.ds(va * EMB, VC * EMB)],
                             sem_w0)
            rb.wait()
            transpose(in1_v, out1_v)
            pltpu.async_copy(out1_v, lin_hbm.at[pl.ds(vb * EMB, VC * EMB)],
                             sem_w1)

        return c

    lax.fori_loop(0, (NPAIR + NW - 1) // NW, pair_body, 0)
    pltpu.make_async_copy(
        out0_v, lin_hbm.at[pl.ds(0, VC * EMB)], sem_w0).wait()
    pltpu.make_async_copy(
        out1_v, lin_hbm.at[pl.ds(0, VC * EMB)], sem_w1).wait()

    @pl.when(wid == 0)
    def _():
        pltpu.sync_copy(tail_hbm, tail_v)
        pltpu.sync_copy(tail_v,
                        lin_hbm.at[pl.ds(VMAIN * EMB, VTAIL * EMB)])


NSG = 8
NBG = 4
S_PER = SEQ // NSG
BG = 4096 // NBG


def _lookup_body(tokT_hbm, map1_hbm, map2_hbm, emb_hbm, pe_hbm, out_hbm,
                 tok_v, t1_v, t2_v, rows1_v, rows2_v, out_v, pe_v,
                 sem_m, sem_e0, sem_e1):
    wid = lax.axis_index("s") * NC + lax.axis_index("c")
    sg = wid // NBG
    bg = wid % NBG
    b0 = bg * BG
    pltpu.sync_copy(pe_hbm.at[pl.ds(sg * S_PER, S_PER)], pe_v)
    iot = lax.iota(jnp.int32, 16)
    sems = (sem_e0, sem_e1)

    def issue_tok_maps(g):
        st = g % 2
        pltpu.sync_copy(tokT_hbm.at[sg * S_PER + g, pl.ds(b0, BG)],
                        tok_v.at[st])
        return (pltpu.async_copy(map1_hbm.at[tok_v.at[st]],
                                 t1_v.at[st], sem_m),
                pltpu.async_copy(map2_hbm.at[tok_v.at[st]],
                                 t2_v.at[st], sem_m))

    def issue_rows(g):
        st = g % 2
        return (pltpu.async_copy(emb_hbm.at[t1_v.at[st]],
                                 rows1_v.at[st], sems[st]),
                pltpu.async_copy(emb_hbm.at[t2_v.at[st]],
                                 rows2_v.at[st], sems[st]))

    def compute_store(g):
        st = g % 2
        st_c = jnp.int32(st)

        def dbody(d, c):
            pes = plsc.load_gather(
                pe_v, [jnp.full((16,), g, jnp.int32),
                       jnp.full((16,), d, jnp.int32)])

            @plsc.parallel_loop(0, BG // 16, 1, unroll=4)
            def _(bv):
                bidx = iot + bv * 16
                dd = jnp.full((16,), d, jnp.int32)
                ss = jnp.full((16,), st_c, jnp.int32)
                g1 = plsc.load_gather(rows1_v, [ss, bidx, dd])
                g2 = plsc.load_gather(rows2_v, [ss, bidx, dd])
                out_v[d, pl.ds(bv * 16, 16)] = (g1 + g2) * 2.0 + pes

            return c

        lax.fori_loop(0, EMB, dbody, 0)
        pltpu.sync_copy(out_v,
                        out_hbm.at[sg * S_PER + g, :, pl.ds(b0, BG)])

    maps = {0: issue_tok_maps(0)}
    for c in maps[0]:
        c.wait()
    rows = {0: issue_rows(0)}
    maps[1] = issue_tok_maps(1)
    for g in range(S_PER):
        if g + 1 < S_PER:
            for c in maps[g + 1]:
                c.wait()
            rows[g + 1] = issue_rows(g + 1)
        if g + 2 < S_PER:
            maps[g + 2] = issue_tok_maps(g + 2)
        for c in rows[g]:
            c.wait()
        compute_store(g)


def kernel(tokens, embedding, positional_embedding, mapping1, mapping2):
    bsz, seqlen = tokens.shape
    assert seqlen == SEQ and bsz == NBG * BG
    assert embedding.shape == (VOCAB, EMB)

    mesh = plsc.VectorSubcoreMesh(core_axis_name="c", subcore_axis_name="s")

    lin_flat = pl.kernel(
        _transpose_body,
        out_type=jax.ShapeDtypeStruct((VOCAB * EMB,), jnp.float32),
        mesh=mesh,
        compiler_params=pltpu.CompilerParams(
            use_tc_tiling_on_sc=True, needs_layout_passes=False),
        scratch_types=[
            pltpu.VMEM((EMB, VC), jnp.float32),
            pltpu.VMEM((EMB, VC), jnp.float32),
            pltpu.VMEM((VC * EMB,), jnp.float32),
            pltpu.VMEM((VC * EMB,), jnp.float32),
            pltpu.VMEM((VTAIL * EMB,), jnp.float32),
            pltpu.SemaphoreType.DMA,
            pltpu.SemaphoreType.DMA,
            pltpu.SemaphoreType.DMA,
        ],
    )(embedding.T, embedding[VMAIN:].reshape(VTAIL * EMB))
    emb_lin = lin_flat.reshape(VOCAB, EMB)

    tokT = tokens.T.astype(jnp.int32)
    pe = positional_embedding.reshape(-1, EMB)

    out = pl.kernel(
        _lookup_body,
        out_type=jax.ShapeDtypeStruct((SEQ, EMB, bsz), jnp.float32),
        mesh=mesh,
        compiler_params=pltpu.CompilerParams(
            use_tc_tiling_on_sc=False, needs_layout_passes=False),
        scratch_types=[
            pltpu.VMEM((2, BG), jnp.int32),
            pltpu.VMEM((2, BG), jnp.int32),
            pltpu.VMEM((2, BG), jnp.int32),
            pltpu.VMEM((2, BG, EMB), jnp.float32),
            pltpu.VMEM((2, BG, EMB), jnp.float32),
            pltpu.VMEM((EMB, BG), jnp.float32),
            pltpu.VMEM((S_PER, EMB), jnp.float32),
            pltpu.SemaphoreType.DMA,
            pltpu.SemaphoreType.DMA,
            pltpu.SemaphoreType.DMA,
        ],
    )(tokT, mapping1, mapping2, emb_lin, pe)
    return out.transpose(2, 0, 1)


# speedup vs baseline: 2.0961x; 1.0238x over previous
import functools

import jax
import jax.numpy as jnp
from jax import lax
from jax.experimental import pallas as pl
from jax.experimental.pallas import tpu as pltpu
from jax.experimental.pallas import tpu_sc as plsc

NC, NS = 2, 16
NW = NC * NS
SEQ = 200
EMB = 16
VOCAB = 1000003
VC = 1024
NVCHUNK = VOCAB // VC
VMAIN = NVCHUNK * VC
VTAIL = VOCAB - VMAIN


NPAIR = NVCHUNK // 2


def _transpose_body(embT_hbm, tail_hbm, lin_hbm,
                    in0_v, in1_v, out0_v, out1_v, tail_v,
                    sem_r, sem_w0, sem_w1):
    wid = lax.axis_index("s") * NC + lax.axis_index("c")
    iot = lax.iota(jnp.int32, 16)

    def transpose(in_v, out_v):
        @plsc.parallel_loop(0, VC, 1, unroll=16)
        def _(v):
            out_v[pl.ds(v * EMB, EMB)] = plsc.load_gather(
                in_v, [iot, jnp.full((16,), v, jnp.int32)]
            )

    def pair_body(k, c):
        p = wid + k * NW

        @pl.when(p < NPAIR)
        def _():
            va = 2 * p * VC
            vb = va + VC

            @pl.when(k > 0)
            def _():
                pltpu.make_async_copy(
                    out0_v, lin_hbm.at[pl.ds(va * EMB, VC * EMB)],
                    sem_w0).wait()
                pltpu.make_async_copy(
                    out1_v, lin_hbm.at[pl.ds(va * EMB, VC * EMB)],
                    sem_w1).wait()

            pltpu.sync_copy(embT_hbm.at[:, pl.ds(va, VC)], in0_v)
            rb = pltpu.async_copy(embT_hbm.at[:, pl.ds(vb, VC)], in1_v,
                                  sem_r)
            transpose(in0_v, out0_v)
            pltpu.async_copy(out0_v, lin_hbm.at[pl.ds(va * EMB, VC * EMB)],
                             sem_w0)
            rb.wait()
            transpose(in1_v, out1_v)
            pltpu.async_copy(out1_v, lin_hbm.at[pl.ds(vb * EMB, VC * EMB)],
                             sem_w1)

        return c

    lax.fori_loop(0, (NPAIR + NW - 1) // NW, pair_body, 0)
    pltpu.make_async_copy(
        out0_v, lin_hbm.at[pl.ds(0, VC * EMB)], sem_w0).wait()
    pltpu.make_async_copy(
        out1_v, lin_hbm.at[pl.ds(0, VC * EMB)], sem_w1).wait()

    @pl.when(wid == 0)
    def _():
        pltpu.sync_copy(tail_hbm, tail_v)
        pltpu.sync_copy(tail_v,
                        lin_hbm.at[pl.ds(VMAIN * EMB, VTAIL * EMB)])


NSG = 8
NBG = 4
S_PER = SEQ // NSG
BG = 4096 // NBG


def _lookup_body(tokT_hbm, map1_hbm, map2_hbm, emb_hbm, pe_hbm, out_hbm,
                 tok_v, t1_v, t2_v, rows1_v, rows2_v, out_v, pe_v,
                 sem_m, sem_e0, sem_e1, sem_o0, sem_o1):
    wid = lax.axis_index("s") * NC + lax.axis_index("c")
    sg = wid // NBG
    bg = wid % NBG
    b0 = bg * BG
    pltpu.sync_copy(pe_hbm.at[pl.ds(sg * S_PER, S_PER)], pe_v)
    iot = lax.iota(jnp.int32, 16)
    sems = (sem_e0, sem_e1)

    def issue_tok_maps(g):
        st = g % 2
        pltpu.sync_copy(tokT_hbm.at[sg * S_PER + g, pl.ds(b0, BG)],
                        tok_v.at[st])
        return (pltpu.async_copy(map1_hbm.at[tok_v.at[st]],
                                 t1_v.at[st], sem_m),
                pltpu.async_copy(map2_hbm.at[tok_v.at[st]],
                                 t2_v.at[st], sem_m))

    def issue_rows(g):
        st = g % 2
        return (pltpu.async_copy(emb_hbm.at[t1_v.at[st]],
                                 rows1_v.at[st], sems[st]),
                pltpu.async_copy(emb_hbm.at[t2_v.at[st]],
                                 rows2_v.at[st], sems[st]))

    osems = (sem_o0, sem_o1)

    def compute_store(g):
        st = g % 2
        st_c = jnp.int32(st)
        if g >= 2:
            pltpu.make_async_copy(
                out_v.at[st],
                out_hbm.at[sg * S_PER + g, :, pl.ds(b0, BG)],
                osems[st]).wait()

        def dbody(d, c):
            pes = plsc.load_gather(
                pe_v, [jnp.full((16,), g, jnp.int32),
                       jnp.full((16,), d, jnp.int32)])

            @plsc.parallel_loop(0, BG // 16, 1, unroll=4)
            def _(bv):
                bidx = iot + bv * 16
                dd = jnp.full((16,), d, jnp.int32)
                ss = jnp.full((16,), st_c, jnp.int32)
                g1 = plsc.load_gather(rows1_v, [ss, bidx, dd])
                g2 = plsc.load_gather(rows2_v, [ss, bidx, dd])
                out_v[st, d, pl.ds(bv * 16, 16)] = (g1 + g2) * 2.0 + pes

            return c

        lax.fori_loop(0, EMB, dbody, 0)
        pltpu.async_copy(out_v.at[st],
                         out_hbm.at[sg * S_PER + g, :, pl.ds(b0, BG)],
                         osems[st])

    maps = {0: issue_tok_maps(0)}
    for c in maps[0]:
        c.wait()
    rows = {0: issue_rows(0)}
    maps[1] = issue_tok_maps(1)
    for g in range(S_PER):
        if g + 1 < S_PER:
            for c in maps[g + 1]:
                c.wait()
            rows[g + 1] = issue_rows(g + 1)
        if g + 2 < S_PER:
            maps[g + 2] = issue_tok_maps(g + 2)
        for c in rows[g]:
            c.wait()
        compute_store(g)
    for st in range(2):
        pltpu.make_async_copy(
            out_v.at[st],
            out_hbm.at[sg * S_PER, :, pl.ds(b0, BG)],
            osems[st]).wait()


def kernel(tokens, embedding, positional_embedding, mapping1, mapping2):
    bsz, seqlen = tokens.shape
    assert seqlen == SEQ and bsz == NBG * BG
    assert embedding.shape == (VOCAB, EMB)

    mesh = plsc.VectorSubcoreMesh(core_axis_name="c", subcore_axis_name="s")

    lin_flat = pl.kernel(
        _transpose_body,
        out_type=jax.ShapeDtypeStruct((VOCAB * EMB,), jnp.float32),
        mesh=mesh,
        compiler_params=pltpu.CompilerParams(
            use_tc_tiling_on_sc=True, needs_layout_passes=False),
        scratch_types=[
            pltpu.VMEM((EMB, VC), jnp.float32),
            pltpu.VMEM((EMB, VC), jnp.float32),
            pltpu.VMEM((VC * EMB,), jnp.float32),
            pltpu.VMEM((VC * EMB,), jnp.float32),
            pltpu.VMEM((VTAIL * EMB,), jnp.float32),
            pltpu.SemaphoreType.DMA,
            pltpu.SemaphoreType.DMA,
            pltpu.SemaphoreType.DMA,
        ],
    )(embedding.T, embedding[VMAIN:].reshape(VTAIL * EMB))
    emb_lin = lin_flat.reshape(VOCAB, EMB)

    tokT = tokens.T.astype(jnp.int32)
    pe = positional_embedding.reshape(-1, EMB)

    out = pl.kernel(
        _lookup_body,
        out_type=jax.ShapeDtypeStruct((SEQ, EMB, bsz), jnp.float32),
        mesh=mesh,
        compiler_params=pltpu.CompilerParams(
            use_tc_tiling_on_sc=False, needs_layout_passes=False),
        scratch_types=[
            pltpu.VMEM((2, BG), jnp.int32),
            pltpu.VMEM((2, BG), jnp.int32),
            pltpu.VMEM((2, BG), jnp.int32),
            pltpu.VMEM((2, BG, EMB), jnp.float32),
            pltpu.VMEM((2, BG, EMB), jnp.float32),
            pltpu.VMEM((2, EMB, BG), jnp.float32),
            pltpu.VMEM((S_PER, EMB), jnp.float32),
            pltpu.SemaphoreType.DMA,
            pltpu.SemaphoreType.DMA,
            pltpu.SemaphoreType.DMA,
            pltpu.SemaphoreType.DMA,
            pltpu.SemaphoreType.DMA,
        ],
    )(tokT, mapping1, mapping2, emb_lin, pe)
    return out.transpose(2, 0, 1)


# speedup vs baseline: 2.0976x; 1.0007x over previous
import jax
import jax.numpy as jnp
from jax import lax
from jax.experimental import pallas as pl
from jax.experimental.pallas import tpu as pltpu
from jax.experimental.pallas import tpu_sc as plsc

NC, NS = 2, 16
NW = NC * NS
SEQ = 200
EMB = 16
VOCAB = 1000003
VC = 1024
NVCHUNK = VOCAB // VC
VMAIN = NVCHUNK * VC
VTAIL = VOCAB - VMAIN


NPAIR = NVCHUNK // 2


def _transpose_body(embT_hbm, tail_hbm, lin_hbm,
                    in0_v, in1_v, out0_v, out1_v, tail_v,
                    sem_r, sem_w0, sem_w1):
    wid = lax.axis_index("s") * NC + lax.axis_index("c")
    iot = lax.iota(jnp.int32, 16)

    def transpose(in_v, out_v):
        @plsc.parallel_loop(0, VC, 1, unroll=16)
        def _(v):
            out_v[pl.ds(v * EMB, EMB)] = plsc.load_gather(
                in_v, [iot, jnp.full((16,), v, jnp.int32)]
            )

    def pair_body(k, c):
        p = wid + k * NW

        @pl.when(p < NPAIR)
        def _():
            va = 2 * p * VC
            vb = va + VC

            @pl.when(k > 0)
            def _():
                pltpu.make_async_copy(
                    out0_v, lin_hbm.at[pl.ds(va * EMB, VC * EMB)],
                    sem_w0).wait()
                pltpu.make_async_copy(
                    out1_v, lin_hbm.at[pl.ds(va * EMB, VC * EMB)],
                    sem_w1).wait()

            pltpu.sync_copy(embT_hbm.at[:, pl.ds(va, VC)], in0_v)
            rb = pltpu.async_copy(embT_hbm.at[:, pl.ds(vb, VC)], in1_v,
                                  sem_r)
            transpose(in0_v, out0_v)
            pltpu.async_copy(out0_v, lin_hbm.at[pl.ds(va * EMB, VC * EMB)],
                             sem_w0)
            rb.wait()
            transpose(in1_v, out1_v)
            pltpu.async_copy(out1_v, lin_hbm.at[pl.ds(vb * EMB, VC * EMB)],
                             sem_w1)

        return c

    lax.fori_loop(0, (NPAIR + NW - 1) // NW, pair_body, 0)
    pltpu.make_async_copy(
        out0_v, lin_hbm.at[pl.ds(0, VC * EMB)], sem_w0).wait()
    pltpu.make_async_copy(
        out1_v, lin_hbm.at[pl.ds(0, VC * EMB)], sem_w1).wait()

    @pl.when(wid == 0)
    def _():
        pltpu.sync_copy(tail_hbm, tail_v)
        pltpu.sync_copy(tail_v,
                        lin_hbm.at[pl.ds(VMAIN * EMB, VTAIL * EMB)])


NSG = 8
NBG = 4
S_PER = SEQ // NSG
BG = 4096 // NBG


def _lookup_body(tokT_hbm, map1_hbm, map2_hbm, emb_hbm, pe_hbm, out_hbm,
                 tok_v, t1_v, t2_v, rows1_v, rows2_v, out_v, pe_v,
                 sem_m, sem_e0, sem_e1, sem_o0, sem_o1):
    wid = lax.axis_index("s") * NC + lax.axis_index("c")
    sg = wid // NBG
    bg = wid % NBG
    b0 = bg * BG
    pltpu.sync_copy(pe_hbm.at[pl.ds(sg * S_PER, S_PER)], pe_v)
    iot = lax.iota(jnp.int32, 16)
    sems = (sem_e0, sem_e1)

    def issue_tok_maps(g):
        st = g % 2
        pltpu.sync_copy(tokT_hbm.at[sg * S_PER + g, pl.ds(b0, BG)],
                        tok_v.at[st])
        return (pltpu.async_copy(map1_hbm.at[tok_v.at[st]],
                                 t1_v.at[st], sem_m),
                pltpu.async_copy(map2_hbm.at[tok_v.at[st]],
                                 t2_v.at[st], sem_m))

    def issue_rows(g):
        st = g % 2
        return (pltpu.async_copy(emb_hbm.at[t1_v.at[st]],
                                 rows1_v.at[st], sems[st]),
                pltpu.async_copy(emb_hbm.at[t2_v.at[st]],
                                 rows2_v.at[st], sems[st]))

    osems = (sem_o0, sem_o1)

    def compute_store(g):
        st = g % 2
        st_c = jnp.int32(st)
        if g >= 2:
            pltpu.make_async_copy(
                out_v.at[st],
                out_hbm.at[sg * S_PER + g, :, pl.ds(b0, BG)],
                osems[st]).wait()

        def dbody(d, c):
            pes = plsc.load_gather(
                pe_v, [jnp.full((16,), g, jnp.int32),
                       jnp.full((16,), d, jnp.int32)])

            @plsc.parallel_loop(0, BG // 16, 1, unroll=4)
            def _(bv):
                bidx = iot + bv * 16
                dd = jnp.full((16,), d, jnp.int32)
                ss = jnp.full((16,), st_c, jnp.int32)
                g1 = plsc.load_gather(rows1_v, [ss, bidx, dd])
                g2 = plsc.load_gather(rows2_v, [ss, bidx, dd])
                out_v[st, d, pl.ds(bv * 16, 16)] = (g1 + g2) * 2.0 + pes

            return c

        lax.fori_loop(0, EMB, dbody, 0)
        pltpu.async_copy(out_v.at[st],
                         out_hbm.at[sg * S_PER + g, :, pl.ds(b0, BG)],
                         osems[st])

    maps = {0: issue_tok_maps(0)}
    for c in maps[0]:
        c.wait()
    rows = {0: issue_rows(0)}
    maps[1] = issue_tok_maps(1)
    for g in range(S_PER):
        if g + 1 < S_PER:
            for c in maps[g + 1]:
                c.wait()
            rows[g + 1] = issue_rows(g + 1)
        if g + 2 < S_PER:
            maps[g + 2] = issue_tok_maps(g + 2)
        for c in rows[g]:
            c.wait()
        compute_store(g)
    for st in range(2):
        pltpu.make_async_copy(
            out_v.at[st],
            out_hbm.at[sg * S_PER, :, pl.ds(b0, BG)],
            osems[st]).wait()


def kernel(tokens, embedding, positional_embedding, mapping1, mapping2):
    bsz, seqlen = tokens.shape
    assert seqlen == SEQ and bsz == NBG * BG
    assert embedding.shape == (VOCAB, EMB)

    mesh = plsc.VectorSubcoreMesh(core_axis_name="c", subcore_axis_name="s")

    lin_flat = pl.kernel(
        _transpose_body,
        out_type=jax.ShapeDtypeStruct((VOCAB * EMB,), jnp.float32),
        mesh=mesh,
        compiler_params=pltpu.CompilerParams(
            use_tc_tiling_on_sc=True, needs_layout_passes=False),
        scratch_types=[
            pltpu.VMEM((EMB, VC), jnp.float32),
            pltpu.VMEM((EMB, VC), jnp.float32),
            pltpu.VMEM((VC * EMB,), jnp.float32),
            pltpu.VMEM((VC * EMB,), jnp.float32),
            pltpu.VMEM((VTAIL * EMB,), jnp.float32),
            pltpu.SemaphoreType.DMA,
            pltpu.SemaphoreType.DMA,
            pltpu.SemaphoreType.DMA,
        ],
    )(embedding.T, embedding[VMAIN:].reshape(VTAIL * EMB))
    emb_lin = lin_flat.reshape(VOCAB, EMB)

    tokT = tokens.T.astype(jnp.int32)
    pe = positional_embedding.reshape(-1, EMB)

    out = pl.kernel(
        _lookup_body,
        out_type=jax.ShapeDtypeStruct((SEQ, EMB, bsz), jnp.float32),
        mesh=mesh,
        compiler_params=pltpu.CompilerParams(
            use_tc_tiling_on_sc=False, needs_layout_passes=False),
        scratch_types=[
            pltpu.VMEM((2, BG), jnp.int32),
            pltpu.VMEM((2, BG), jnp.int32),
            pltpu.VMEM((2, BG), jnp.int32),
            pltpu.VMEM((2, BG, EMB), jnp.float32),
            pltpu.VMEM((2, BG, EMB), jnp.float32),
            pltpu.VMEM((2, EMB, BG), jnp.float32),
            pltpu.VMEM((S_PER, EMB), jnp.float32),
            pltpu.SemaphoreType.DMA,
            pltpu.SemaphoreType.DMA,
            pltpu.SemaphoreType.DMA,
            pltpu.SemaphoreType.DMA,
            pltpu.SemaphoreType.DMA,
        ],
    )(tokT, mapping1, mapping2, emb_lin, pe)
    return out.transpose(2, 0, 1)
